# baseline (speedup 1.0000x reference)
import functools

import jax
import jax.numpy as jnp
from jax import lax
from jax.experimental import pallas as pl
from jax.experimental.pallas import tpu as pltpu
from jax.experimental.pallas import tpu_sc as plsc

NC, NS = 2, 16
K = 125
NQ = 2
NPAD = 10240
RPT = NPAD // NS
ZROWS = 128


def _sc_segment_sums(xl, xr, src3, dst3, et3, rell, relr):
    n, hd = xl.shape
    r = rell.shape[0]
    nchunk = src3.shape[1]

    mesh = plsc.VectorSubcoreMesh(
        core_axis_name="c", subcore_axis_name="s", num_cores=NC, num_subcores=NS
    )

    @functools.partial(
        pl.kernel,
        out_type=(
            jax.ShapeDtypeStruct((NPAD, hd), jnp.float32),
            jax.ShapeDtypeStruct((NPAD, hd), jnp.float32),
            jax.ShapeDtypeStruct((2, NPAD), jnp.float32),
        ),
        mesh=mesh,
        scratch_types=[
            pltpu.VMEM((nchunk // NQ, K), jnp.int32),
            pltpu.VMEM((nchunk // NQ, K), jnp.int32),
            pltpu.VMEM((nchunk // NQ, K), jnp.int32),
            pltpu.VMEM((K, hd), jnp.float32),
            pltpu.VMEM((K, hd), jnp.float32),
            pltpu.VMEM((K, hd), jnp.float32),
            pltpu.VMEM((K, hd), jnp.float32),
            pltpu.VMEM((128,), jnp.float32),
            pltpu.VMEM((ZROWS, hd), jnp.float32),
            pltpu.VMEM((RPT,), jnp.float32),
            pltpu.VMEM_SHARED((NPAD, hd), jnp.float32),
            pltpu.VMEM_SHARED((NPAD,), jnp.float32),
            pltpu.VMEM_SHARED((r, hd), jnp.float32),
            pltpu.SemaphoreType.DMA,
            pltpu.SemaphoreType.DMA,
            pltpu.SemaphoreType.DMA,
            pltpu.SemaphoreType.DMA,
            pltpu.SemaphoreType.DMA,
            pltpu.SemaphoreType.DMA,
            pltpu.SemaphoreType.DMA,
            pltpu.SemaphoreType.DMA,
        ],
        compiler_params=pltpu.CompilerParams(use_tc_tiling_on_sc=False),
    )
    def sc_fn(xl_hbm, xr_hbm, src_hbm, dst_hbm, et_hbm, rell_hbm, relr_hbm,
              s0_hbm, s1_hbm, deg_hbm,
              srcv, dstv, etv, xb0, xb1, rb0, rb1, ones, zbuf, dstage,
              acc, degacc, srel,
              gsem0, gsem1, gsem2, gsem3, ssem0, ssem1, dsem0, dsem1):
        c = lax.axis_index("c")
        s = lax.axis_index("s")

        zv = jnp.zeros((16,), jnp.float32)
        ov = jnp.ones((16,), jnp.float32)
        nsub = hd // 16

        def zrow(i, _):
            for g in range(nsub):
                zbuf[i, pl.ds(g * 16, 16)] = zv
            return 0

        lax.fori_loop(0, ZROWS, zrow, 0)

        def zdeg(i, _):
            dstage[pl.ds(i * 16, 16)] = zv
            return 0

        lax.fori_loop(0, RPT // 16, zdeg, 0)
        for g in range(8):
            ones[pl.ds(g * 16, 16)] = ov

        for i in range(RPT // ZROWS):
            pltpu.sync_copy(zbuf, acc.at[pl.ds(s * RPT + i * ZROWS, ZROWS)])
        pltpu.sync_copy(dstage, degacc.at[pl.ds(s * RPT, RPT)])

        @pl.when(jnp.logical_and(s == 0, c == 0))
        def _():
            pltpu.sync_copy(rell_hbm, srel)

        @pl.when(jnp.logical_and(s == 0, c == 1))
        def _():
            pltpu.sync_copy(relr_hbm, srel)

        plsc.subcore_barrier()

        def addbuf(dref, sref):
            @plsc.parallel_loop(0, K, 1, unroll=5)
            def _(i):
                for g in range(nsub):
                    sl = pl.ds(g * 16, 16)
                    dref[i, sl] = dref[i, sl] + sref[i, sl]

        def gx(j, buf, sem):
            @pl.when(c == 0)
            def _():
                pltpu.async_copy(xl_hbm.at[srcv.at[j]], buf, sem)

            @pl.when(c == 1)
            def _():
                pltpu.async_copy(xr_hbm.at[srcv.at[j]], buf, sem)

        def wait_gx(j, buf, sem):
            @pl.when(c == 0)
            def _():
                pltpu.make_async_copy(xl_hbm.at[srcv.at[j]], buf, sem).wait()

            @pl.when(c == 1)
            def _():
                pltpu.make_async_copy(xr_hbm.at[srcv.at[j]], buf, sem).wait()

        def gr(j, buf, sem):
            pltpu.async_copy(srel.at[etv.at[j]], buf, sem)

        def wait_gr(j, buf, sem):
            pltpu.make_async_copy(srel.at[etv.at[j]], buf, sem).wait()

        def sx(j, buf, sem):
            pltpu.async_copy(buf, acc.at[dstv.at[j]], sem, add=True)

        def wait_sx(j, buf, sem):
            pltpu.make_async_copy(buf, acc.at[dstv.at[j]], sem).wait()

        def sd(j, sem, deg_on):
            @pl.when(deg_on)
            def _():
                pltpu.async_copy(ones.at[pl.ds(0, K)],
                                 degacc.at[dstv.at[j]], sem, add=True)

        def wait_sd(j, sem, deg_on):
            @pl.when(deg_on)
            def _():
                pltpu.make_async_copy(ones.at[pl.ds(0, K)],
                                      degacc.at[dstv.at[j]], sem).wait()

        qchunk = nchunk // NQ
        npair = qchunk // 2
        for q in range(NQ):
            deg_on = (c == 0) if q < NQ // 2 else (c == 1)
            qs = pl.ds(q * qchunk, qchunk)
            pltpu.sync_copy(src_hbm.at[s, qs], srcv)
            pltpu.sync_copy(dst_hbm.at[s, qs], dstv)
            pltpu.sync_copy(et_hbm.at[s, qs], etv)

            def body(jj, peel_first, fire_next):
                j0 = 2 * jj
                j1 = j0 + 1
                if not peel_first:
                    wait_sx(j1 - 2, xb1, ssem1)
                    wait_sd(j1 - 2, dsem1, deg_on)
                gx(j1, xb1, gsem2)
                wait_gx(j0, xb0, gsem0)
                wait_gr(j0, rb0, gsem1)
                addbuf(xb0, rb0)
                if fire_next:
                    gr(j0 + 2, rb0, gsem1)
                sx(j0, xb0, ssem0)
                sd(j0, dsem0, deg_on)
                wait_gx(j1, xb1, gsem2)
                wait_gr(j1, rb1, gsem3)
                addbuf(xb1, rb1)
                if fire_next:
                    gr(j1 + 2, rb1, gsem3)
                wait_sx(j0, xb0, ssem0)
                wait_sd(j0, dsem0, deg_on)
                if fire_next:
                    gx(j0 + 2, xb0, gsem0)
                sx(j1, xb1, ssem1)
                sd(j1, dsem1, deg_on)

            gx(0, xb0, gsem0)
            gr(0, rb0, gsem1)
            gr(1, rb1, gsem3)
            body(0, True, True)

            def mid(jj, _):
                body(jj, False, True)
                return 0

            lax.fori_loop(1, npair - 1, mid, 0)
            body(npair - 1, False, False)
            wait_sx(qchunk - 1, xb1, ssem1)
            wait_sd(qchunk - 1, dsem1, deg_on)

        plsc.subcore_barrier()

        @pl.when(c == 0)
        def _():
            for i in range(RPT // ZROWS):
                rows = pl.ds(s * RPT + i * ZROWS, ZROWS)
                pltpu.sync_copy(acc.at[rows], zbuf)
                pltpu.sync_copy(zbuf, s0_hbm.at[rows])

        @pl.when(c == 1)
        def _():
            for i in range(RPT // ZROWS):
                rows = pl.ds(s * RPT + i * ZROWS, ZROWS)
                pltpu.sync_copy(acc.at[rows], zbuf)
                pltpu.sync_copy(zbuf, s1_hbm.at[rows])

        pltpu.sync_copy(degacc.at[pl.ds(s * RPT, RPT)], dstage)
        pltpu.sync_copy(dstage, deg_hbm.at[c, pl.ds(s * RPT, RPT)])

    return sc_fn(xl, xr, src3, dst3, et3, rell, relr)


def _tc_combine(s0, s1, x, norm, deg0, deg1, wn, wl, we):
    n, d = x.shape
    hd = d // 2
    bs = 512

    def body(s0_ref, s1_ref, x_ref, norm_ref, d0_ref, d1_ref,
             wn_ref, wl_ref, we_ref, o_ref):
        h = jnp.dot(s0_ref[...], wn_ref[0:hd, :],
                    preferred_element_type=jnp.float32)
        h = h + jnp.dot(s1_ref[...], wn_ref[hd:d, :],
                        preferred_element_type=jnp.float32)
        xb = x_ref[...]
        lm_main = jnp.dot(xb, wl_ref[...], preferred_element_type=jnp.float32)
        lm_evo = jnp.dot(xb, we_ref[...], preferred_element_type=jnp.float32)
        deg = d0_ref[...] + d1_ref[...]
        o_ref[...] = h * norm_ref[...] + jnp.where(
            deg > 0.0, lm_main, lm_evo)

    half_spec = pl.BlockSpec((bs, hd), lambda i: (i, 0))
    row_spec = pl.BlockSpec((bs, d), lambda i: (i, 0))
    col_spec = pl.BlockSpec((bs, 1), lambda i: (i, 0))
    w_spec = pl.BlockSpec((d, d), lambda i: (0, 0))

    return pl.pallas_call(
        body,
        grid=(NPAD // bs,),
        in_specs=[half_spec, half_spec, row_spec, col_spec, col_spec,
                  col_spec, w_spec, w_spec, w_spec],
        out_specs=row_spec,
        out_shape=jax.ShapeDtypeStruct((n, d), jnp.float32),
    )(s0, s1, x, norm, deg0, deg1, wn, wl, we)


def kernel(x, edge_index, edge_type, norm, emb_rel, prev_h,
           weight_neighbor, loop_weight, evolve_loop_weight):
    n, d = x.shape
    e = edge_type.shape[0]
    hd = d // 2
    nchunk = e // (NS * K)
    assert e == NS * K * nchunk and n <= NPAD

    src3 = edge_index[0].reshape(NS, nchunk, K)
    dst3 = edge_index[1].reshape(NS, nchunk, K)
    et3 = edge_type.reshape(NS, nchunk, K)
    xl = x[:, :hd]
    xr = x[:, hd:]
    rell = emb_rel[:, :hd]
    relr = emb_rel[:, hd:]

    s0, s1, degp = _sc_segment_sums(xl, xr, src3, dst3, et3, rell, relr)
    deg0 = degp[0].reshape(NPAD, 1)
    deg1 = degp[1].reshape(NPAD, 1)
    return _tc_combine(s0, s1, x, norm, deg0, deg1,
                       weight_neighbor, loop_weight, evolve_loop_weight)


# speedup vs baseline: 1.0461x; 1.0461x over previous
import functools

import jax
import jax.numpy as jnp
from jax import lax
from jax.experimental import pallas as pl
from jax.experimental.pallas import tpu as pltpu
from jax.experimental.pallas import tpu_sc as plsc

NC, NS = 2, 16
K = 125
NQ = 2
NPAD = 10240
RPT = NPAD // NS
ZROWS = 128


def _sc_segment_sums(xl, xr, ei3, et3, rell, relr):
    n, hd = xl.shape
    r = rell.shape[0]
    nchunk = ei3.shape[2]

    mesh = plsc.VectorSubcoreMesh(
        core_axis_name="c", subcore_axis_name="s", num_cores=NC, num_subcores=NS
    )

    @functools.partial(
        pl.kernel,
        out_type=(
            jax.ShapeDtypeStruct((NPAD, hd), jnp.float32),
            jax.ShapeDtypeStruct((NPAD, hd), jnp.float32),
            jax.ShapeDtypeStruct((2, NPAD), jnp.float32),
        ),
        mesh=mesh,
        scratch_types=[
            pltpu.VMEM((nchunk // NQ, K), jnp.int32),
            pltpu.VMEM((nchunk // NQ, K), jnp.int32),
            pltpu.VMEM((nchunk // NQ, K), jnp.int32),
            pltpu.VMEM((K, hd), jnp.float32),
            pltpu.VMEM((K, hd), jnp.float32),
            pltpu.VMEM((K, hd), jnp.float32),
            pltpu.VMEM((K, hd), jnp.float32),
            pltpu.VMEM((128,), jnp.float32),
            pltpu.VMEM((ZROWS, hd), jnp.float32),
            pltpu.VMEM((RPT,), jnp.float32),
            pltpu.VMEM_SHARED((NPAD, hd), jnp.float32),
            pltpu.VMEM_SHARED((NPAD,), jnp.float32),
            pltpu.VMEM_SHARED((r, hd), jnp.float32),
            pltpu.SemaphoreType.DMA,
            pltpu.SemaphoreType.DMA,
            pltpu.SemaphoreType.DMA,
            pltpu.SemaphoreType.DMA,
            pltpu.SemaphoreType.DMA,
            pltpu.SemaphoreType.DMA,
            pltpu.SemaphoreType.DMA,
            pltpu.SemaphoreType.DMA,
        ],
        compiler_params=pltpu.CompilerParams(use_tc_tiling_on_sc=False),
    )
    def sc_fn(xl_hbm, xr_hbm, ei_hbm, et_hbm, rell_hbm, relr_hbm,
              s0_hbm, s1_hbm, deg_hbm,
              srcv, dstv, etv, xb0, xb1, rb0, rb1, ones, zbuf, dstage,
              acc, degacc, srel,
              gsem0, gsem1, gsem2, gsem3, ssem0, ssem1, dsem0, dsem1):
        c = lax.axis_index("c")
        s = lax.axis_index("s")

        zv = jnp.zeros((16,), jnp.float32)
        ov = jnp.ones((16,), jnp.float32)
        nsub = hd // 16

        def zrow(i, _):
            for g in range(nsub):
                zbuf[i, pl.ds(g * 16, 16)] = zv
            return 0

        lax.fori_loop(0, ZROWS, zrow, 0)

        def zdeg(i, _):
            dstage[pl.ds(i * 16, 16)] = zv
            return 0

        lax.fori_loop(0, RPT // 16, zdeg, 0)
        for g in range(8):
            ones[pl.ds(g * 16, 16)] = ov

        for i in range(RPT // ZROWS):
            pltpu.sync_copy(zbuf, acc.at[pl.ds(s * RPT + i * ZROWS, ZROWS)])
        pltpu.sync_copy(dstage, degacc.at[pl.ds(s * RPT, RPT)])

        @pl.when(jnp.logical_and(s == 0, c == 0))
        def _():
            pltpu.sync_copy(rell_hbm, srel)

        @pl.when(jnp.logical_and(s == 0, c == 1))
        def _():
            pltpu.sync_copy(relr_hbm, srel)

        plsc.subcore_barrier()

        def addbuf(dref, sref):
            @plsc.parallel_loop(0, K, 1, unroll=5)
            def _(i):
                for g in range(nsub):
                    sl = pl.ds(g * 16, 16)
                    dref[i, sl] = dref[i, sl] + sref[i, sl]

        def gx(j, buf, sem):
            @pl.when(c == 0)
            def _():
                pltpu.async_copy(xl_hbm.at[srcv.at[j]], buf, sem)

            @pl.when(c == 1)
            def _():
                pltpu.async_copy(xr_hbm.at[srcv.at[j]], buf, sem)

        def wait_gx(j, buf, sem):
            @pl.when(c == 0)
            def _():
                pltpu.make_async_copy(xl_hbm.at[srcv.at[j]], buf, sem).wait()

            @pl.when(c == 1)
            def _():
                pltpu.make_async_copy(xr_hbm.at[srcv.at[j]], buf, sem).wait()

        def gr(j, buf, sem):
            pltpu.async_copy(srel.at[etv.at[j]], buf, sem)

        def wait_gr(j, buf, sem):
            pltpu.make_async_copy(srel.at[etv.at[j]], buf, sem).wait()

        def sx(j, buf, sem):
            pltpu.async_copy(buf, acc.at[dstv.at[j]], sem, add=True)

        def wait_sx(j, buf, sem):
            pltpu.make_async_copy(buf, acc.at[dstv.at[j]], sem).wait()

        def sd(j, sem, deg_on):
            @pl.when(deg_on)
            def _():
                pltpu.async_copy(ones.at[pl.ds(0, K)],
                                 degacc.at[dstv.at[j]], sem, add=True)

        def wait_sd(j, sem, deg_on):
            @pl.when(deg_on)
            def _():
                pltpu.make_async_copy(ones.at[pl.ds(0, K)],
                                      degacc.at[dstv.at[j]], sem).wait()

        qchunk = nchunk // NQ
        npair = qchunk // 2
        for q in range(NQ):
            deg_on = (c == 0) if q < NQ // 2 else (c == 1)
            qs = pl.ds(q * qchunk, qchunk)
            pltpu.sync_copy(ei_hbm.at[0, s, qs], srcv)
            pltpu.sync_copy(ei_hbm.at[1, s, qs], dstv)
            pltpu.sync_copy(et_hbm.at[s, qs], etv)

            def body(jj, peel_first, fire_next):
                j0 = 2 * jj
                j1 = j0 + 1
                if not peel_first:
                    wait_sx(j1 - 2, xb1, ssem1)
                    wait_sd(j1 - 2, dsem1, deg_on)
                gx(j1, xb1, gsem2)
                wait_gx(j0, xb0, gsem0)
                wait_gr(j0, rb0, gsem1)
                addbuf(xb0, rb0)
                if fire_next:
                    gr(j0 + 2, rb0, gsem1)
                sx(j0, xb0, ssem0)
                sd(j0, dsem0, deg_on)
                wait_gx(j1, xb1, gsem2)
                wait_gr(j1, rb1, gsem3)
                addbuf(xb1, rb1)
                if fire_next:
                    gr(j1 + 2, rb1, gsem3)
                wait_sx(j0, xb0, ssem0)
                wait_sd(j0, dsem0, deg_on)
                if fire_next:
                    gx(j0 + 2, xb0, gsem0)
                sx(j1, xb1, ssem1)
                sd(j1, dsem1, deg_on)

            gx(0, xb0, gsem0)
            gr(0, rb0, gsem1)
            gr(1, rb1, gsem3)
            body(0, True, True)

            def mid(jj, _):
                body(jj, False, True)
                return 0

            lax.fori_loop(1, npair - 1, mid, 0)
            body(npair - 1, False, False)
            wait_sx(qchunk - 1, xb1, ssem1)
            wait_sd(qchunk - 1, dsem1, deg_on)

        plsc.subcore_barrier()

        @pl.when(c == 0)
        def _():
            for i in range(RPT // ZROWS):
                rows = pl.ds(s * RPT + i * ZROWS, ZROWS)
                pltpu.sync_copy(acc.at[rows], zbuf)
                pltpu.sync_copy(zbuf, s0_hbm.at[rows])

        @pl.when(c == 1)
        def _():
            for i in range(RPT // ZROWS):
                rows = pl.ds(s * RPT + i * ZROWS, ZROWS)
                pltpu.sync_copy(acc.at[rows], zbuf)
                pltpu.sync_copy(zbuf, s1_hbm.at[rows])

        pltpu.sync_copy(degacc.at[pl.ds(s * RPT, RPT)], dstage)
        pltpu.sync_copy(dstage, deg_hbm.at[c, pl.ds(s * RPT, RPT)])

    return sc_fn(xl, xr, ei3, et3, rell, relr)


def _tc_combine(s0, s1, x, norm, degp3, wn, wl, we):
    n, d = x.shape
    hd = d // 2
    bs = 1000

    def body(s0_ref, s1_ref, x_ref, norm_ref, d0_ref, d1_ref,
             wn_ref, wl_ref, we_ref, o_ref):
        h = jnp.dot(s0_ref[...], wn_ref[0:hd, :],
                    preferred_element_type=jnp.float32)
        h = h + jnp.dot(s1_ref[...], wn_ref[hd:d, :],
                        preferred_element_type=jnp.float32)
        xb = x_ref[...]
        lm_main = jnp.dot(xb, wl_ref[...], preferred_element_type=jnp.float32)
        lm_evo = jnp.dot(xb, we_ref[...], preferred_element_type=jnp.float32)
        deg = d0_ref[...][0] + d1_ref[...][0]
        o_ref[...] = h * norm_ref[...] + jnp.where(
            deg > 0.0, lm_main, lm_evo)

    half_spec = pl.BlockSpec((bs, hd), lambda i: (i, 0))
    row_spec = pl.BlockSpec((bs, d), lambda i: (i, 0))
    col_spec = pl.BlockSpec((bs, 1), lambda i: (i, 0))
    d0_spec = pl.BlockSpec((1, bs, 1), lambda i: (0, i, 0))
    d1_spec = pl.BlockSpec((1, bs, 1), lambda i: (1, i, 0))
    w_spec = pl.BlockSpec((d, d), lambda i: (0, 0))

    return pl.pallas_call(
        body,
        grid=(n // bs,),
        in_specs=[half_spec, half_spec, row_spec, col_spec, d0_spec,
                  d1_spec, w_spec, w_spec, w_spec],
        out_specs=row_spec,
        out_shape=jax.ShapeDtypeStruct((n, d), jnp.float32),
    )(s0, s1, x, norm, degp3, degp3, wn, wl, we)


def kernel(x, edge_index, edge_type, norm, emb_rel, prev_h,
           weight_neighbor, loop_weight, evolve_loop_weight):
    n, d = x.shape
    e = edge_type.shape[0]
    hd = d // 2
    nchunk = e // (NS * K)
    assert e == NS * K * nchunk and n <= NPAD

    ei3 = edge_index.reshape(2, NS, nchunk, K)
    et3 = edge_type.reshape(NS, nchunk, K)
    xl = x[:, :hd]
    xr = x[:, hd:]
    rell = emb_rel[:, :hd]
    relr = emb_rel[:, hd:]

    s0, s1, degp = _sc_segment_sums(xl, xr, ei3, et3, rell, relr)
    return _tc_combine(s0, s1, x, norm, degp.reshape(2, NPAD, 1),
                       weight_neighbor, loop_weight, evolve_loop_weight)
